# baseline (speedup 1.0000x reference)
import functools

import jax
import jax.numpy as jnp
from jax import lax
from jax.experimental import pallas as pl
from jax.experimental.pallas import tpu as pltpu
from jax.experimental.pallas import tpu_sc as plsc

D = 64
V = 1000000
NB = 16384
NF = 26
NC, NS = 2, 16
NW = NC * NS
SUB = 128
NJ = NB // SUB
NT = NF * NJ
T_PER_W = NT // NW
L = 16
VP = V // 2
NBLK = V // SUB
BLK_I = (NBLK + NW - 1) // NW

_mesh = plsc.VectorSubcoreMesh(core_axis_name="c", subcore_axis_name="s")


@functools.partial(
    pl.kernel,
    mesh=_mesh,
    compiler_params=pltpu.CompilerParams(needs_layout_passes=False),
    out_type=jax.ShapeDtypeStruct((VP, 2 * D), jnp.float32),
    scratch_types=[
        pltpu.VMEM((D, SUB + 1), jnp.float32),
        pltpu.VMEM((D, SUB + 1), jnp.float32),
        pltpu.VMEM((SUB // 2, 2 * D), jnp.float32),
        pltpu.VMEM((SUB // 2, 2 * D), jnp.float32),
        pltpu.SemaphoreType.DMA,
        pltpu.SemaphoreType.DMA,
        pltpu.SemaphoreType.DMA,
        pltpu.SemaphoreType.DMA,
    ],
)
def _transpose_kernel(tt_hbm, tail_hbm, t2_hbm, vbuf_a, vbuf_b, obuf_a,
                      obuf_b, rsem_a, rsem_b, wsem_a, wsem_b):
    wid = lax.axis_index("s") * NC + lax.axis_index("c")

    def blk(slot):
        return wid + NW * slot

    def fire_read(slot, vbuf, rsem):
        j = blk(slot)

        @pl.when(j < NBLK)
        def _():
            pltpu.async_copy(tt_hbm.at[:, pl.ds(j * SUB, SUB)],
                             vbuf.at[:, pl.ds(0, SUB)], rsem)

    def drain_read(vbuf, rsem):
        pltpu.make_async_copy(tt_hbm.at[:, pl.ds(0, SUB)],
                              vbuf.at[:, pl.ds(0, SUB)], rsem).wait()

    dvs = [jnp.arange(L, dtype=jnp.int32) + d0 for d0 in range(0, D, L)]

    def transpose(vbuf, obuf):
        def q_body(q, carry):
            kv0 = jnp.full((L,), 0, jnp.int32) + 2 * q
            kv1 = kv0 + 1
            gs = []
            for i in range(D // L):
                gs.append(plsc.load_gather(vbuf, [dvs[i], kv0]))
                gs.append(plsc.load_gather(vbuf, [dvs[i], kv1]))
            for i in range(D // L):
                obuf[q, pl.ds(i * L, L)] = gs[2 * i]
                obuf[q, pl.ds(D + i * L, L)] = gs[2 * i + 1]
            return carry

        lax.fori_loop(0, SUB // 2, q_body, 0)

    def fire_write(slot, obuf, wsem):
        j = blk(slot)

        @pl.when(j < NBLK)
        def _():
            pltpu.async_copy(obuf, t2_hbm.at[pl.ds(j * (SUB // 2), SUB // 2)],
                             wsem)

    def drain_write(obuf, wsem):
        pltpu.make_async_copy(obuf, t2_hbm.at[pl.ds(0, SUB // 2)], wsem).wait()

    fire_read(0, vbuf_a, rsem_a)
    fire_read(1, vbuf_b, rsem_b)

    def round_body(r, carry):
        sa = 2 * r
        sb = sa + 1

        @pl.when(jnp.logical_and(r > 0, blk(sa - 2) < NBLK))
        def _():
            drain_write(obuf_a, wsem_a)

        @pl.when(blk(sa) < NBLK)
        def _():
            drain_read(vbuf_a, rsem_a)
            transpose(vbuf_a, obuf_a)
            fire_read(sa + 2, vbuf_a, rsem_a)
            fire_write(sa, obuf_a, wsem_a)

        @pl.when(jnp.logical_and(r > 0, blk(sb - 2) < NBLK))
        def _():
            drain_write(obuf_b, wsem_b)

        @pl.when(blk(sb) < NBLK)
        def _():
            drain_read(vbuf_b, rsem_b)
            transpose(vbuf_b, obuf_b)
            fire_read(sb + 2, vbuf_b, rsem_b)
            fire_write(sb, obuf_b, wsem_b)
        return carry

    lax.fori_loop(0, (BLK_I + 1) // 2, round_body, 0)

    @pl.when(blk(2 * ((BLK_I + 1) // 2) - 2) < NBLK)
    def _():
        drain_write(obuf_a, wsem_a)

    @pl.when(blk(2 * ((BLK_I + 1) // 2) - 1) < NBLK)
    def _():
        drain_write(obuf_b, wsem_b)

    @pl.when(wid == 0)
    def _():
        pltpu.sync_copy(tail_hbm, vbuf_a.at[:, pl.ds(0, SUB)])

        def tq_body(q, carry):
            kv0 = jnp.full((L,), 0, jnp.int32) + 2 * q
            kv1 = kv0 + 1
            for i in range(D // L):
                obuf_a[q, pl.ds(i * L, L)] = plsc.load_gather(
                    vbuf_a, [dvs[i], kv0])
                obuf_a[q, pl.ds(D + i * L, L)] = plsc.load_gather(
                    vbuf_a, [dvs[i], kv1])
            return carry

        lax.fori_loop(0, D // 2, tq_body, 0)
        pltpu.sync_copy(obuf_a.at[pl.ds(0, D // 2)],
                        t2_hbm.at[pl.ds(NBLK * (SUB // 2), D // 2)])


@functools.partial(
    pl.kernel,
    mesh=_mesh,
    compiler_params=pltpu.CompilerParams(needs_layout_passes=False),
    out_type=jax.ShapeDtypeStruct((NF, D, NB), jnp.float32),
    scratch_types=(
        [pltpu.VMEM((SUB,), jnp.int32) for _ in range(3)]
        + [pltpu.VMEM((SUB,), jnp.int32) for _ in range(3)]
        + [pltpu.VMEM((SUB, 2 * D + 1), jnp.float32) for _ in range(3)]
        + [pltpu.VMEM((D, SUB), jnp.float32) for _ in range(3)]
        + [pltpu.SemaphoreType.DMA] * 9
    ),
)
def _gather_kernel(ctx_hbm, table_hbm, out_hbm,
                   idx0, idx1, idx2, pid0, pid1, pid2,
                   pb0, pb1, pb2, tb0, tb1, tb2,
                   is0, is1, is2, gs0, gs1, gs2, ws0, ws1, ws2):
    idx_r = (idx0, idx1, idx2)
    pid_r = (pid0, pid1, pid2)
    pb_r = (pb0, pb1, pb2)
    tb_r = (tb0, tb1, tb2)
    is_r = (is0, is1, is2)
    gs_r = (gs0, gs1, gs2)
    ws_r = (ws0, ws1, ws2)

    wid = lax.axis_index("s") * NC + lax.axis_index("c")
    t0 = wid * T_PER_W

    def fire_idx(t, s):
        f = t >> 7
        j = t & (NJ - 1)
        pltpu.async_copy(ctx_hbm.at[f, pl.ds(j * SUB, SUB)], idx_r[s],
                         is_r[s])

    def wait_idx(s):
        pltpu.make_async_copy(ctx_hbm.at[0, pl.ds(0, SUB)], idx_r[s],
                              is_r[s]).wait()

    def fire_gather(s):
        for m in range(SUB // L):
            pid_r[s][pl.ds(m * L, L)] = lax.shift_right_logical(
                idx_r[s][pl.ds(m * L, L)], 1)
        pltpu.async_copy(table_hbm.at[pid_r[s]],
                         pb_r[s].at[:, pl.ds(0, 2 * D)], gs_r[s])

    def wait_gather(s):
        pltpu.make_async_copy(table_hbm.at[pl.ds(0, SUB)],
                              pb_r[s].at[:, pl.ds(0, 2 * D)], gs_r[s]).wait()

    def extract(s):
        idx_v, pbuf, tbuf = idx_r[s], pb_r[s], tb_r[s]
        rowss, h64s = [], []
        for m in range(SUB // L):
            rowss.append(jnp.arange(L, dtype=jnp.int32) + m * L)
            h64s.append(lax.shift_left(
                lax.bitwise_and(idx_v[pl.ds(m * L, L)], 1), 6))
        def d_body(dd, carry):
            d0 = 2 * dd
            gs0 = [plsc.load_gather(pbuf, [rowss[m], h64s[m] + d0])
                   for m in range(SUB // L)]
            gs1 = [plsc.load_gather(pbuf, [rowss[m], h64s[m] + (d0 + 1)])
                   for m in range(SUB // L)]
            for m in range(SUB // L):
                tbuf[d0, pl.ds(m * L, L)] = gs0[m]
            for m in range(SUB // L):
                tbuf[d0 + 1, pl.ds(m * L, L)] = gs1[m]
            return carry

        lax.fori_loop(0, D // 2, d_body, 0)

    def fire_write(t, s):
        f = t >> 7
        j = t & (NJ - 1)
        pltpu.async_copy(tb_r[s], out_hbm.at[f, :, pl.ds(j * SUB, SUB)],
                         ws_r[s])

    def wait_write(s):
        pltpu.make_async_copy(tb_r[s], out_hbm.at[0, :, pl.ds(0, SUB)],
                              ws_r[s]).wait()

    fire_idx(t0, 0)
    fire_idx(t0 + 1, 1)
    fire_idx(t0 + 2, 2)
    wait_idx(0)
    fire_gather(0)
    wait_idx(1)
    fire_gather(1)

    def step(i, carry):
        t = t0 + i
        s = lax.rem(i, 3)
        for sv in range(3):
            @pl.when(s == sv)
            def _():
                s2 = (sv + 2) % 3

                @pl.when(i + 2 < T_PER_W)
                def _():
                    wait_idx(s2)
                    fire_gather(s2)
                wait_gather(sv)

                @pl.when(i >= 3)
                def _():
                    wait_write(sv)
                extract(sv)
                fire_write(t, sv)

                @pl.when(i + 3 < T_PER_W)
                def _():
                    fire_idx(t + 3, sv)
        return carry

    lax.fori_loop(0, T_PER_W, step, 0)
    wait_write(2)
    wait_write(0)
    wait_write(1)


def kernel(context, table):
    tail = jnp.pad(table[NBLK * SUB:], ((0, SUB - D), (0, 0))).T
    t2 = _transpose_kernel(table.T, tail)
    ctx_t = context.T
    out = _gather_kernel(ctx_t, t2)
    return out.transpose(2, 0, 1)


# speedup vs baseline: 1.5602x; 1.5602x over previous
import functools

import jax
import jax.numpy as jnp
from jax import lax
from jax.experimental import pallas as pl
from jax.experimental.pallas import tpu as pltpu
from jax.experimental.pallas import tpu_sc as plsc

D = 64
B = 16384 * 26
NC, NS = 2, 16
NW = NC * NS
SUB = 128
R = B // SUB
R_PER_W = R // NW
NSTR = 4
RPR = 2 * NSTR
NR = R_PER_W // RPR

_mesh = plsc.VectorSubcoreMesh(core_axis_name="c", subcore_axis_name="s")


@functools.partial(
    pl.kernel,
    mesh=_mesh,
    compiler_params=pltpu.CompilerParams(use_tc_tiling_on_sc=False),
    out_type=jax.ShapeDtypeStruct((R, SUB, D), jnp.float32),
    scratch_types=[
        pltpu.VMEM((R_PER_W, SUB), jnp.int32),
        pltpu.VMEM((NSTR, SUB, D), jnp.float32),
        pltpu.VMEM((NSTR, SUB, D), jnp.float32),
        pltpu.SemaphoreType.DMA,
        pltpu.SemaphoreType.DMA,
    ],
)
def _gather_kernel(idx_hbm, table_hbm, out_hbm, idx_v, buf_a, buf_b, sem_a, sem_b):
    wid = lax.axis_index("s") * NC + lax.axis_index("c")
    base = wid * R_PER_W

    pltpu.sync_copy(idx_hbm.at[pl.ds(base, R_PER_W)], idx_v)

    def fire(buf, sem, row0):
        for j in range(NSTR):
            pltpu.async_copy(table_hbm.at[idx_v.at[row0 + j]], buf.at[j], sem)

    def drain(buf, sem):
        for j in range(NSTR):
            pltpu.make_async_copy(table_hbm.at[pl.ds(0, SUB)], buf.at[j], sem).wait()

    fire(buf_a, sem_a, 0)

    def round_body(r, carry):
        row_a = r * RPR
        row_b = row_a + NSTR
        fire(buf_b, sem_b, row_b)
        drain(buf_a, sem_a)
        pltpu.sync_copy(buf_a, out_hbm.at[pl.ds(base + row_a, NSTR)])

        @pl.when(r < NR - 1)
        def _():
            fire(buf_a, sem_a, row_a + RPR)

        drain(buf_b, sem_b)
        pltpu.sync_copy(buf_b, out_hbm.at[pl.ds(base + row_b, NSTR)])
        return carry

    lax.fori_loop(0, NR, round_body, 0)


def kernel(context, table):
    idx2 = context.reshape(R, SUB)
    out = _gather_kernel(idx2, table)
    return out.reshape(context.shape[0], context.shape[1], D)
